# baseline (speedup 1.0000x reference)
import functools

import jax
import jax.numpy as jnp
from jax import lax
from jax.experimental import pallas as pl
from jax.experimental.pallas import tpu as pltpu
from jax.experimental.pallas import tpu_sc as plsc

N = 10000
D = 128
E = 320000
N_CLASSES = 40

NC = 2
NS = 16
NW = NC * NS
EPT = E // NW
C = 96
NCHUNK = 108
EPT_PAD = NCHUNK * C
N_ACC = N + 8
ROWS_PT = 624
ROWS_LAST = N - 15 * ROWS_PT

_mesh = plsc.VectorSubcoreMesh(core_axis_name="c", subcore_axis_name="s")


@functools.partial(
    pl.kernel,
    mesh=_mesh,
    out_type=[
        jax.ShapeDtypeStruct((N, D), jnp.float32),
        jax.ShapeDtypeStruct((N, D), jnp.float32),
    ],
    scratch_types=[
        pltpu.VMEM((C,), jnp.int32),
        pltpu.VMEM((C,), jnp.int32),
        pltpu.VMEM((C,), jnp.int32),
        pltpu.VMEM((C,), jnp.int32),
        pltpu.VMEM((C,), jnp.int32),
        pltpu.VMEM((C,), jnp.int32),
        pltpu.VMEM((C,), jnp.int32),
        pltpu.VMEM((C,), jnp.int32),
        pltpu.VMEM((C, D), jnp.float32),
        pltpu.VMEM((C, D), jnp.float32),
        pltpu.VMEM((C, D), jnp.float32),
        pltpu.VMEM((C, D), jnp.float32),
        pltpu.VMEM_SHARED((N_ACC, D), jnp.float32),
        pltpu.SemaphoreType.DMA,
        pltpu.SemaphoreType.DMA,
        pltpu.SemaphoreType.DMA,
        pltpu.SemaphoreType.DMA,
    ],
)
def _sc_agg(x_hbm, src_hbm, dst_hbm, zeros_hbm, out0, out1,
            si0, si1, si2, si3, di0, di1, di2, di3,
            rows0, rows1, rows2, rows3, acc, sem_g, sem_s, sem_i, sem_d):
    c = lax.axis_index("c")
    s = lax.axis_index("s")
    wid = c * NS + s
    r0 = s * ROWS_PT
    base = wid * EPT_PAD
    sbufs = [si0, si1, si2, si3]
    dbufs = [di0, di1, di2, di3]
    rbufs = [rows0, rows1, rows2, rows3]

    def _sidx_copy(g, buf):
        return pltpu.make_async_copy(src_hbm.at[pl.ds(base + g * C, C)],
                                     buf, sem_i)

    def _didx_copy(g, buf):
        return pltpu.make_async_copy(dst_hbm.at[pl.ds(base + g * C, C)],
                                     buf, sem_d)

    def _scat_drain():
        pltpu.make_async_copy(rows0, acc.at[di0], sem_s).wait()

    @pl.when(s < NS - 1)
    def _():
        pltpu.async_copy(zeros_hbm.at[pl.ds(r0, ROWS_PT)],
                         acc.at[pl.ds(r0, ROWS_PT)], sem_s)

    @pl.when(s == NS - 1)
    def _():
        pltpu.async_copy(zeros_hbm.at[pl.ds(r0, ROWS_LAST)],
                         acc.at[pl.ds(r0, ROWS_LAST)], sem_s)

    for j in range(4):
        _sidx_copy(j, sbufs[j]).start()
    _didx_copy(0, di0).start()
    _didx_copy(1, di1).start()
    _sidx_copy(0, si0).wait()
    pltpu.async_copy(x_hbm.at[si0], rows0, sem_g)
    _sidx_copy(1, si1).wait()
    pltpu.async_copy(x_hbm.at[si1], rows1, sem_g)

    @pl.when(s < NS - 1)
    def _():
        pltpu.make_async_copy(zeros_hbm.at[pl.ds(r0, ROWS_PT)],
                              acc.at[pl.ds(r0, ROWS_PT)], sem_s).wait()

    @pl.when(s == NS - 1)
    def _():
        pltpu.make_async_copy(zeros_hbm.at[pl.ds(r0, ROWS_LAST)],
                              acc.at[pl.ds(r0, ROWS_LAST)], sem_s).wait()

    plsc.subcore_barrier()

    def _chunk_step(g, j):
        @pl.when(g >= 2)
        def _():
            _scat_drain()

        @pl.when(g + 2 < NCHUNK)
        def _():
            _didx_copy(g + 2, dbufs[(j + 2) % 4]).start()
            _sidx_copy(g + 2, sbufs[(j + 2) % 4]).wait()
            pltpu.async_copy(x_hbm.at[sbufs[(j + 2) % 4]], rbufs[(j + 2) % 4],
                             sem_g)
        pltpu.make_async_copy(x_hbm.at[sbufs[j]], rbufs[j], sem_g).wait()

        @pl.when(g + 4 < NCHUNK)
        def _():
            _sidx_copy(g + 4, sbufs[j]).start()
        _didx_copy(g, dbufs[j]).wait()
        pltpu.async_copy(rbufs[j], acc.at[dbufs[j]], sem_s, add=True)

    def body(t, carry):
        for j in range(4):
            _chunk_step(4 * t + j, j)
        return carry

    lax.fori_loop(0, NCHUNK // 4, body, 0)
    _scat_drain()
    _scat_drain()
    plsc.subcore_barrier()

    out = [out0, out1]
    for ci in range(NC):
        @pl.when((c == ci) & (s < NS - 1))
        def _(ci=ci):
            pltpu.sync_copy(acc.at[pl.ds(r0, ROWS_PT)],
                            out[ci].at[pl.ds(r0, ROWS_PT)])

        @pl.when((c == ci) & (s == NS - 1))
        def _(ci=ci):
            pltpu.sync_copy(acc.at[pl.ds(r0, ROWS_LAST)],
                            out[ci].at[pl.ds(r0, ROWS_LAST)])


ROWS_BLK = 1000


def _mlp_body(x_ref, p0_ref, p1_ref, w_ref, b_ref, o_ref):
    z = x_ref[...] + p0_ref[...] + p1_ref[...]
    h = jnp.dot(z, w_ref[...], preferred_element_type=jnp.float32) + b_ref[...]
    o_ref[...] = jnp.maximum(h, 0.0)


def _tc_mlp(x, p0, p1, W, b):
    return pl.pallas_call(
        _mlp_body,
        grid=(N // ROWS_BLK,),
        in_specs=[
            pl.BlockSpec((ROWS_BLK, D), lambda i: (i, 0)),
            pl.BlockSpec((ROWS_BLK, D), lambda i: (i, 0)),
            pl.BlockSpec((ROWS_BLK, D), lambda i: (i, 0)),
            pl.BlockSpec((D, D), lambda i: (0, 0)),
            pl.BlockSpec((1, D), lambda i: (0, 0)),
        ],
        out_specs=pl.BlockSpec((ROWS_BLK, D), lambda i: (i, 0)),
        out_shape=jax.ShapeDtypeStruct((N, D), jnp.float32),
    )(x, p0, p1, W, b.reshape(1, D))


def _final_body(x_ref, p0_ref, p1_ref, w2_ref, b2_ref, wc_ref, bc_ref, o_ref):
    z = x_ref[...] + p0_ref[...] + p1_ref[...]
    h = jnp.dot(z, w2_ref[...], preferred_element_type=jnp.float32) + b2_ref[...]
    h = jnp.maximum(h, 0.0)
    o_ref[...] = jnp.dot(h, wc_ref[...], preferred_element_type=jnp.float32) + bc_ref[...]


def _tc_final(x, p0, p1, W2, b2, Wc, bc):
    return pl.pallas_call(
        _final_body,
        grid=(N // ROWS_BLK,),
        in_specs=[
            pl.BlockSpec((ROWS_BLK, D), lambda i: (i, 0)),
            pl.BlockSpec((ROWS_BLK, D), lambda i: (i, 0)),
            pl.BlockSpec((ROWS_BLK, D), lambda i: (i, 0)),
            pl.BlockSpec((D, D), lambda i: (0, 0)),
            pl.BlockSpec((1, D), lambda i: (0, 0)),
            pl.BlockSpec((D, N_CLASSES), lambda i: (0, 0)),
            pl.BlockSpec((1, N_CLASSES), lambda i: (0, 0)),
        ],
        out_specs=pl.BlockSpec((ROWS_BLK, N_CLASSES), lambda i: (i, 0)),
        out_shape=jax.ShapeDtypeStruct((N, N_CLASSES), jnp.float32),
    )(x, p0, p1, W2, b2.reshape(1, D), Wc, bc.reshape(1, N_CLASSES))


def kernel(feat, edge_index, W0, b0, W1, b1, W2, b2, Wc, bc):
    src = jnp.pad(edge_index[0].astype(jnp.int32).reshape(NW, EPT),
                  ((0, 0), (0, EPT_PAD - EPT))).reshape(-1)
    junk = N + jnp.tile(jnp.arange(8, dtype=jnp.int32),
                        (EPT_PAD - EPT) // 8 + 1)[:EPT_PAD - EPT]
    dst = jnp.concatenate(
        [edge_index[1].astype(jnp.int32).reshape(NW, EPT),
         jnp.broadcast_to(junk, (NW, EPT_PAD - EPT))], axis=1).reshape(-1)
    zeros = jnp.zeros((N, D), jnp.float32)
    p0, p1 = _sc_agg(feat, src, dst, zeros)
    h = _tc_mlp(feat, p0, p1, W0, b0)
    p0, p1 = _sc_agg(h, src, dst, zeros)
    h = _tc_mlp(h, p0, p1, W1, b1)
    p0, p1 = _sc_agg(h, src, dst, zeros)
    return _tc_final(h, p0, p1, W2, b2, Wc, bc)


# speedup vs baseline: 4.7199x; 4.7199x over previous
import functools

import jax
import jax.numpy as jnp
from jax import lax
from jax.experimental import pallas as pl
from jax.experimental.pallas import tpu as pltpu
from jax.experimental.pallas import tpu_sc as plsc

N = 10000
D = 128
E = 320000
N_CLASSES = 40

NC = 2
NS = 16
NW = NC * NS
EPT = E // NW
C = 96
NCHUNK = 108
EPT_PAD = NCHUNK * C
N_ACC = N + 8
ROWS_PT = 624
ROWS_LAST = N - 15 * ROWS_PT

_mesh = plsc.VectorSubcoreMesh(core_axis_name="c", subcore_axis_name="s")


@functools.partial(
    pl.kernel,
    mesh=_mesh,
    out_type=[
        jax.ShapeDtypeStruct((N, D), jnp.float32),
        jax.ShapeDtypeStruct((N, D), jnp.float32),
    ],
    scratch_types=[
        pltpu.VMEM((C,), jnp.int32),
        pltpu.VMEM((C,), jnp.int32),
        pltpu.VMEM((C,), jnp.int32),
        pltpu.VMEM((C,), jnp.int32),
        pltpu.VMEM((C,), jnp.int32),
        pltpu.VMEM((C,), jnp.int32),
        pltpu.VMEM((C,), jnp.int32),
        pltpu.VMEM((C,), jnp.int32),
        pltpu.VMEM((C, D), jnp.float32),
        pltpu.VMEM((C, D), jnp.float32),
        pltpu.VMEM((C, D), jnp.float32),
        pltpu.VMEM((C, D), jnp.float32),
        pltpu.VMEM_SHARED((N_ACC, D), jnp.float32),
        pltpu.SemaphoreType.DMA,
        pltpu.SemaphoreType.DMA,
        pltpu.SemaphoreType.DMA,
        pltpu.SemaphoreType.DMA,
    ],
)
def _sc_agg(x_hbm, src_hbm, dst_hbm, zeros_hbm, out0, out1,
            si0, si1, si2, si3, di0, di1, di2, di3,
            rows0, rows1, rows2, rows3, acc, sem_g, sem_s, sem_i, sem_d):
    c = lax.axis_index("c")
    s = lax.axis_index("s")
    wid = c * NS + s
    r0 = s * ROWS_PT
    base = wid * EPT_PAD
    sbufs = [si0, si1, si2, si3]
    dbufs = [di0, di1, di2, di3]
    rbufs = [rows0, rows1, rows2, rows3]

    def _sidx_copy(g, buf):
        return pltpu.make_async_copy(src_hbm.at[pl.ds(base + g * C, C)],
                                     buf, sem_i)

    def _didx_copy(g, buf):
        return pltpu.make_async_copy(dst_hbm.at[pl.ds(base + g * C, C)],
                                     buf, sem_d)

    def _scat_drain():
        pltpu.make_async_copy(rows0, acc.at[di0], sem_s).wait()

    @pl.when(s < NS - 1)
    def _():
        pltpu.async_copy(zeros_hbm.at[pl.ds(r0, ROWS_PT)],
                         acc.at[pl.ds(r0, ROWS_PT)], sem_s)

    @pl.when(s == NS - 1)
    def _():
        pltpu.async_copy(zeros_hbm.at[pl.ds(r0, ROWS_LAST)],
                         acc.at[pl.ds(r0, ROWS_LAST)], sem_s)

    for j in range(4):
        _sidx_copy(j, sbufs[j]).start()
    _didx_copy(0, di0).start()
    _didx_copy(1, di1).start()
    _sidx_copy(0, si0).wait()
    pltpu.async_copy(x_hbm.at[si0], rows0, sem_g)
    _sidx_copy(1, si1).wait()
    pltpu.async_copy(x_hbm.at[si1], rows1, sem_g)

    @pl.when(s < NS - 1)
    def _():
        pltpu.make_async_copy(zeros_hbm.at[pl.ds(r0, ROWS_PT)],
                              acc.at[pl.ds(r0, ROWS_PT)], sem_s).wait()

    @pl.when(s == NS - 1)
    def _():
        pltpu.make_async_copy(zeros_hbm.at[pl.ds(r0, ROWS_LAST)],
                              acc.at[pl.ds(r0, ROWS_LAST)], sem_s).wait()

    plsc.subcore_barrier()

    def _chunk_step(g, j):
        @pl.when(g >= 2)
        def _():
            _scat_drain()

        @pl.when(g + 2 < NCHUNK)
        def _():
            _didx_copy(g + 2, dbufs[(j + 2) % 4]).start()
            _sidx_copy(g + 2, sbufs[(j + 2) % 4]).wait()
            pltpu.async_copy(x_hbm.at[sbufs[(j + 2) % 4]], rbufs[(j + 2) % 4],
                             sem_g)
        pltpu.make_async_copy(x_hbm.at[sbufs[j]], rbufs[j], sem_g).wait()

        @pl.when(g + 4 < NCHUNK)
        def _():
            _sidx_copy(g + 4, sbufs[j]).start()
        _didx_copy(g, dbufs[j]).wait()
        pltpu.async_copy(rbufs[j], acc.at[dbufs[j]], sem_s, add=True)

    def body(t, carry):
        for j in range(4):
            _chunk_step(4 * t + j, j)
        return carry

    lax.fori_loop(0, NCHUNK // 4, body, 0)
    _scat_drain()
    _scat_drain()
    plsc.subcore_barrier()

    out = [out0, out1]
    for ci in range(NC):
        @pl.when((c == ci) & (s < NS - 1))
        def _(ci=ci):
            pltpu.sync_copy(acc.at[pl.ds(r0, ROWS_PT)],
                            out[ci].at[pl.ds(r0, ROWS_PT)])

        @pl.when((c == ci) & (s == NS - 1))
        def _(ci=ci):
            pltpu.sync_copy(acc.at[pl.ds(r0, ROWS_LAST)],
                            out[ci].at[pl.ds(r0, ROWS_LAST)])


ROWS_BLK = 1000


def _mlp_body(x_ref, p0_ref, p1_ref, w_ref, b_ref, o_ref):
    z = x_ref[...] + p0_ref[...] + p1_ref[...]
    h = jnp.dot(z, w_ref[...], preferred_element_type=jnp.float32) + b_ref[...]
    o_ref[...] = jnp.maximum(h, 0.0)


def _tc_mlp(x, p0, p1, W, b):
    return pl.pallas_call(
        _mlp_body,
        grid=(N // ROWS_BLK,),
        in_specs=[
            pl.BlockSpec((ROWS_BLK, D), lambda i: (i, 0)),
            pl.BlockSpec((ROWS_BLK, D), lambda i: (i, 0)),
            pl.BlockSpec((ROWS_BLK, D), lambda i: (i, 0)),
            pl.BlockSpec((D, D), lambda i: (0, 0)),
            pl.BlockSpec((1, D), lambda i: (0, 0)),
        ],
        out_specs=pl.BlockSpec((ROWS_BLK, D), lambda i: (i, 0)),
        out_shape=jax.ShapeDtypeStruct((N, D), jnp.float32),
    )(x, p0, p1, W, b.reshape(1, D))


def _final_body(x_ref, p0_ref, p1_ref, w2_ref, b2_ref, wc_ref, bc_ref, o_ref):
    z = x_ref[...] + p0_ref[...] + p1_ref[...]
    h = jnp.dot(z, w2_ref[...], preferred_element_type=jnp.float32) + b2_ref[...]
    h = jnp.maximum(h, 0.0)
    o_ref[...] = jnp.dot(h, wc_ref[...], preferred_element_type=jnp.float32) + bc_ref[...]


def _tc_final(x, p0, p1, W2, b2, Wc, bc):
    return pl.pallas_call(
        _final_body,
        grid=(N // ROWS_BLK,),
        in_specs=[
            pl.BlockSpec((ROWS_BLK, D), lambda i: (i, 0)),
            pl.BlockSpec((ROWS_BLK, D), lambda i: (i, 0)),
            pl.BlockSpec((ROWS_BLK, D), lambda i: (i, 0)),
            pl.BlockSpec((D, D), lambda i: (0, 0)),
            pl.BlockSpec((1, D), lambda i: (0, 0)),
            pl.BlockSpec((D, N_CLASSES), lambda i: (0, 0)),
            pl.BlockSpec((1, N_CLASSES), lambda i: (0, 0)),
        ],
        out_specs=pl.BlockSpec((ROWS_BLK, N_CLASSES), lambda i: (i, 0)),
        out_shape=jax.ShapeDtypeStruct((N, N_CLASSES), jnp.float32),
    )(x, p0, p1, W2, b2.reshape(1, D), Wc, bc.reshape(1, N_CLASSES))


def kernel(feat, edge_index, W0, b0, W1, b1, W2, b2, Wc, bc):
    pad_n = EPT_PAD - EPT
    junk_src = (jnp.arange(NW, dtype=jnp.int32)[:, None] * 331
                + jnp.arange(pad_n, dtype=jnp.int32)[None, :] * 17) % N
    src = jnp.concatenate(
        [edge_index[0].astype(jnp.int32).reshape(NW, EPT), junk_src],
        axis=1).reshape(-1)
    junk = N + jnp.tile(jnp.arange(8, dtype=jnp.int32),
                        (EPT_PAD - EPT) // 8 + 1)[:EPT_PAD - EPT]
    dst = jnp.concatenate(
        [edge_index[1].astype(jnp.int32).reshape(NW, EPT),
         jnp.broadcast_to(junk, (NW, EPT_PAD - EPT))], axis=1).reshape(-1)
    zeros = jnp.zeros((N, D), jnp.float32)
    p0, p1 = _sc_agg(feat, src, dst, zeros)
    h = _tc_mlp(feat, p0, p1, W0, b0)
    p0, p1 = _sc_agg(h, src, dst, zeros)
    h = _tc_mlp(h, p0, p1, W1, b1)
    p0, p1 = _sc_agg(h, src, dst, zeros)
    return _tc_final(h, p0, p1, W2, b2, Wc, bc)


# speedup vs baseline: 4.8146x; 1.0201x over previous
import functools

import jax
import jax.numpy as jnp
from jax import lax
from jax.experimental import pallas as pl
from jax.experimental.pallas import tpu as pltpu
from jax.experimental.pallas import tpu_sc as plsc

N = 10000
D = 128
E = 320000
N_CLASSES = 40

NC = 2
NS = 16
NW = NC * NS
EPT = E // NW
C = 96
NCHUNK = 108
EPT_PAD = NCHUNK * C
N_ACC = N + 8
ROWS_PT = 624
ROWS_LAST = N - 15 * ROWS_PT

_mesh = plsc.VectorSubcoreMesh(core_axis_name="c", subcore_axis_name="s")


@functools.partial(
    pl.kernel,
    mesh=_mesh,
    out_type=[
        jax.ShapeDtypeStruct((N, D), jnp.float32),
        jax.ShapeDtypeStruct((N, D), jnp.float32),
    ],
    scratch_types=[
        pltpu.VMEM((C,), jnp.int32),
        pltpu.VMEM((C,), jnp.int32),
        pltpu.VMEM((C,), jnp.int32),
        pltpu.VMEM((C,), jnp.int32),
        pltpu.VMEM((C,), jnp.int32),
        pltpu.VMEM((C,), jnp.int32),
        pltpu.VMEM((C,), jnp.int32),
        pltpu.VMEM((C,), jnp.int32),
        pltpu.VMEM((C, D), jnp.float32),
        pltpu.VMEM((C, D), jnp.float32),
        pltpu.VMEM((C, D), jnp.float32),
        pltpu.VMEM((C, D), jnp.float32),
        pltpu.VMEM_SHARED((N_ACC, D), jnp.float32),
        pltpu.SemaphoreType.DMA,
        pltpu.SemaphoreType.DMA,
        pltpu.SemaphoreType.DMA,
        pltpu.SemaphoreType.DMA,
    ],
)
def _sc_agg(x_hbm, src_hbm, dst_hbm, zeros_hbm, out0, out1,
            si0, si1, si2, si3, di0, di1, di2, di3,
            rows0, rows1, rows2, rows3, acc, sem_g, sem_s, sem_i, sem_d):
    c = lax.axis_index("c")
    s = lax.axis_index("s")
    wid = c * NS + s
    r0 = s * ROWS_PT
    base = wid * EPT_PAD
    sbufs = [si0, si1, si2, si3]
    dbufs = [di0, di1, di2, di3]
    rbufs = [rows0, rows1, rows2, rows3]

    def _sidx_copy(g, buf):
        return pltpu.make_async_copy(src_hbm.at[pl.ds(base + g * C, C)],
                                     buf, sem_i)

    def _didx_copy(g, buf):
        return pltpu.make_async_copy(dst_hbm.at[pl.ds(base + g * C, C)],
                                     buf, sem_d)

    def _scat_drain():
        pltpu.make_async_copy(rows0, acc.at[di0], sem_s).wait()

    @pl.when(s < NS - 1)
    def _():
        pltpu.async_copy(zeros_hbm.at[pl.ds(r0, ROWS_PT)],
                         acc.at[pl.ds(r0, ROWS_PT)], sem_s)

    @pl.when(s == NS - 1)
    def _():
        pltpu.async_copy(zeros_hbm.at[pl.ds(r0, ROWS_LAST)],
                         acc.at[pl.ds(r0, ROWS_LAST)], sem_s)

    for j in range(4):
        _sidx_copy(j, sbufs[j]).start()
    _didx_copy(0, di0).start()
    _didx_copy(1, di1).start()
    _sidx_copy(0, si0).wait()
    pltpu.async_copy(x_hbm.at[si0], rows0, sem_g)
    _sidx_copy(1, si1).wait()
    pltpu.async_copy(x_hbm.at[si1], rows1, sem_g)

    @pl.when(s < NS - 1)
    def _():
        pltpu.make_async_copy(zeros_hbm.at[pl.ds(r0, ROWS_PT)],
                              acc.at[pl.ds(r0, ROWS_PT)], sem_s).wait()

    @pl.when(s == NS - 1)
    def _():
        pltpu.make_async_copy(zeros_hbm.at[pl.ds(r0, ROWS_LAST)],
                              acc.at[pl.ds(r0, ROWS_LAST)], sem_s).wait()

    plsc.subcore_barrier()

    def _chunk_step(g, j):
        @pl.when(g >= 2)
        def _():
            _scat_drain()

        @pl.when(g + 2 < NCHUNK)
        def _():
            _sidx_copy(g + 2, sbufs[(j + 2) % 4]).wait()
            pltpu.async_copy(x_hbm.at[sbufs[(j + 2) % 4]], rbufs[(j + 2) % 4],
                             sem_g)
            _didx_copy(g + 2, dbufs[(j + 2) % 4]).start()
        pltpu.make_async_copy(x_hbm.at[sbufs[j]], rbufs[j], sem_g).wait()

        @pl.when(g + 4 < NCHUNK)
        def _():
            _sidx_copy(g + 4, sbufs[j]).start()
        _didx_copy(g, dbufs[j]).wait()
        pltpu.async_copy(rbufs[j], acc.at[dbufs[j]], sem_s, add=True)

    def body(t, carry):
        for j in range(4):
            _chunk_step(4 * t + j, j)
        return carry

    lax.fori_loop(0, NCHUNK // 4, body, 0)
    _scat_drain()
    _scat_drain()
    plsc.subcore_barrier()

    out = [out0, out1]
    for ci in range(NC):
        @pl.when((c == ci) & (s < NS - 1))
        def _(ci=ci):
            pltpu.sync_copy(acc.at[pl.ds(r0, ROWS_PT)],
                            out[ci].at[pl.ds(r0, ROWS_PT)])

        @pl.when((c == ci) & (s == NS - 1))
        def _(ci=ci):
            pltpu.sync_copy(acc.at[pl.ds(r0, ROWS_LAST)],
                            out[ci].at[pl.ds(r0, ROWS_LAST)])


ROWS_BLK = 2000


def _mlp_body(x_ref, p0_ref, p1_ref, w_ref, b_ref, o_ref):
    z = x_ref[...] + p0_ref[...] + p1_ref[...]
    h = jnp.dot(z, w_ref[...], preferred_element_type=jnp.float32) + b_ref[...]
    o_ref[...] = jnp.maximum(h, 0.0)


def _tc_mlp(x, p0, p1, W, b):
    return pl.pallas_call(
        _mlp_body,
        grid=(N // ROWS_BLK,),
        in_specs=[
            pl.BlockSpec((ROWS_BLK, D), lambda i: (i, 0)),
            pl.BlockSpec((ROWS_BLK, D), lambda i: (i, 0)),
            pl.BlockSpec((ROWS_BLK, D), lambda i: (i, 0)),
            pl.BlockSpec((D, D), lambda i: (0, 0)),
            pl.BlockSpec((1, D), lambda i: (0, 0)),
        ],
        out_specs=pl.BlockSpec((ROWS_BLK, D), lambda i: (i, 0)),
        out_shape=jax.ShapeDtypeStruct((N, D), jnp.float32),
    )(x, p0, p1, W, b.reshape(1, D))


def _final_body(x_ref, p0_ref, p1_ref, w2_ref, b2_ref, wc_ref, bc_ref, o_ref):
    z = x_ref[...] + p0_ref[...] + p1_ref[...]
    h = jnp.dot(z, w2_ref[...], preferred_element_type=jnp.float32) + b2_ref[...]
    h = jnp.maximum(h, 0.0)
    o_ref[...] = jnp.dot(h, wc_ref[...], preferred_element_type=jnp.float32) + bc_ref[...]


def _tc_final(x, p0, p1, W2, b2, Wc, bc):
    return pl.pallas_call(
        _final_body,
        grid=(N // ROWS_BLK,),
        in_specs=[
            pl.BlockSpec((ROWS_BLK, D), lambda i: (i, 0)),
            pl.BlockSpec((ROWS_BLK, D), lambda i: (i, 0)),
            pl.BlockSpec((ROWS_BLK, D), lambda i: (i, 0)),
            pl.BlockSpec((D, D), lambda i: (0, 0)),
            pl.BlockSpec((1, D), lambda i: (0, 0)),
            pl.BlockSpec((D, N_CLASSES), lambda i: (0, 0)),
            pl.BlockSpec((1, N_CLASSES), lambda i: (0, 0)),
        ],
        out_specs=pl.BlockSpec((ROWS_BLK, N_CLASSES), lambda i: (i, 0)),
        out_shape=jax.ShapeDtypeStruct((N, N_CLASSES), jnp.float32),
    )(x, p0, p1, W2, b2.reshape(1, D), Wc, bc.reshape(1, N_CLASSES))


def kernel(feat, edge_index, W0, b0, W1, b1, W2, b2, Wc, bc):
    pad_n = EPT_PAD - EPT
    junk_src = (jnp.arange(NW, dtype=jnp.int32)[:, None] * 331
                + jnp.arange(pad_n, dtype=jnp.int32)[None, :] * 17) % N
    src = jnp.concatenate(
        [edge_index[0].astype(jnp.int32).reshape(NW, EPT), junk_src],
        axis=1).reshape(-1)
    junk = N + jnp.tile(jnp.arange(8, dtype=jnp.int32),
                        (EPT_PAD - EPT) // 8 + 1)[:EPT_PAD - EPT]
    dst = jnp.concatenate(
        [edge_index[1].astype(jnp.int32).reshape(NW, EPT),
         jnp.broadcast_to(junk, (NW, EPT_PAD - EPT))], axis=1).reshape(-1)
    zeros = jnp.zeros((N, D), jnp.float32)
    p0, p1 = _sc_agg(feat, src, dst, zeros)
    h = _tc_mlp(feat, p0, p1, W0, b0)
    p0, p1 = _sc_agg(h, src, dst, zeros)
    h = _tc_mlp(h, p0, p1, W1, b1)
    p0, p1 = _sc_agg(h, src, dst, zeros)
    return _tc_final(h, p0, p1, W2, b2, Wc, bc)
